# baseline (speedup 1.0000x reference)
import functools
import jax
import jax.numpy as jnp
import numpy as np
from jax import lax
from jax.experimental import pallas as pl
from jax.experimental.pallas import tpu as pltpu
from jax.experimental.pallas import tpu_sc as plsc

_B = 4096
_NS = 26
_ND = 13
_V = 100000
_D = 4
_NF = _NS + _ND
_EPS = 0.001
_PAD = 128

_fi, _fj = np.triu_indices(_NF, k=1)
_FI = _fi.astype(np.int32)
_FJ = _fj.astype(np.int32)

_NW = 32
_NWORDS = _B * _NS * _D
_CHUNK = 128
_WPW = _NWORDS // _NW
_CPW = _WPW // _CHUNK
_FIRE = 13


@functools.lru_cache(maxsize=None)
def _make_sc_gather():
    return functools.partial(
        pl.kernel,
        out_type=jax.ShapeDtypeStruct((_NWORDS,), jnp.float32),
        mesh=plsc.VectorSubcoreMesh(core_axis_name="c", subcore_axis_name="s"),
        scratch_types=[
            pltpu.VMEM((_CPW, _CHUNK), jnp.int32),
            pltpu.VMEM((_WPW,), jnp.float32),
            pltpu.SemaphoreType.DMA,
        ],
        compiler_params=pltpu.CompilerParams(use_tc_tiling_on_sc=False),
    )(_sc_gather_body)


def _sc_gather_body(table_hbm, idx_hbm, out_hbm, idx_v, rows_v, sem):
    wid = lax.axis_index("s") * 2 + lax.axis_index("c")
    pltpu.sync_copy(idx_hbm.at[wid], idx_v)
    for g in range(_CPW // _FIRE):
        cps = []
        for j in range(_FIRE):
            jj = g * _FIRE + j
            cps.append(
                pltpu.async_copy(
                    table_hbm.at[idx_v.at[jj]],
                    rows_v.at[pl.ds(jj * _CHUNK, _CHUNK)],
                    sem,
                )
            )
        for cp in cps:
            cp.wait()
    pltpu.sync_copy(rows_v, out_hbm.at[pl.ds(wid * _WPW, _WPW)])


def _dotNT(a, b):
    return lax.dot_general(
        a, b, (((1,), (1,)), ((), ())),
        preferred_element_type=jnp.float32,
        precision=lax.Precision.HIGHEST,
    )


def _tc_body(es_ref, xdt_ref, w_ref, bmat_ref, out_ref):
    dpt = xdt_ref[...] * w_ref[...]
    zpad = jnp.zeros((_PAD - _NF, _B), jnp.float32)
    f = [jnp.concatenate([es_ref[d], dpt, zpad], axis=0) for d in range(_D)]
    s = _dotNT(f[0], f[0])
    for d in range(1, _D):
        s = s + _dotNT(f[d], f[d])
    m2 = s * (1.0 / _B)
    q = None
    for d in range(_D):
        for d2 in range(d, _D):
            g = f[d] * f[d2]
            t = _dotNT(g, g)
            t = t if d == d2 else t * 2.0
            q = t if q is None else q + t
    var = q * (1.0 / _B) - m2 * m2
    w = jnp.tanh(bmat_ref[...]) * lax.rsqrt(var + _EPS)
    c = jnp.sum(w * m2)
    acc = None
    for d in range(_D):
        a = lax.dot_general(
            w, f[d], (((0,), (0,)), ((), ())),
            preferred_element_type=jnp.float32,
            precision=lax.Precision.HIGHEST,
        ) * f[d]
        acc = a if acc is None else acc + a
    out_ref[...] = jnp.sum(acc, axis=0, keepdims=True) - c


_tc_call = pl.pallas_call(
    _tc_body,
    out_shape=jax.ShapeDtypeStruct((1, _B), jnp.float32),
    compiler_params=pltpu.CompilerParams(vmem_limit_bytes=100 * 1024 * 1024),
)


def kernel(X, emb_tables, weight, beta):
    sparse_idx_t = X[:, :_NS].astype(jnp.int32).T
    table_flat = emb_tables.transpose(0, 2, 1).reshape(_NS * _D * _V)
    widx = (sparse_idx_t[None, :, :]
            + (jnp.arange(_D, dtype=jnp.int32) * _V)[:, None, None]
            + (jnp.arange(_NS, dtype=jnp.int32) * (_D * _V))[None, :, None]
            ).reshape(_NW, _CPW, _CHUNK)
    es_flat = _make_sc_gather()(table_flat, widx)
    es = es_flat.reshape(_D, _NS, _B)
    xdt = X[:, _NS:].T
    bmat = jnp.zeros((_PAD, _PAD), jnp.float32).at[_FI, _FJ].set(beta)
    out = _tc_call(es, xdt, weight, bmat)
    return out.reshape(_B, 1)


# speedup vs baseline: 1.6320x; 1.6320x over previous
import functools
import jax
import jax.numpy as jnp
import numpy as np
from jax import lax
from jax.experimental import pallas as pl
from jax.experimental.pallas import tpu as pltpu
from jax.experimental.pallas import tpu_sc as plsc

_B = 4096
_NS = 26
_ND = 13
_V = 100000
_D = 4
_NF = _NS + _ND
_EPS = 0.001
_PAD = 128

_fi, _fj = np.triu_indices(_NF, k=1)
_P = _fi.shape[0]
_BPAD = 768
_BIDX = np.full((_BPAD,), 127 * 128 + 127, np.int32)
_BIDX[:_P] = (_fi * 128 + _fj).astype(np.int32)
_BIDX = _BIDX.reshape(6, 128)

_NW = 32
_NWORDS = _B * _NS * _D
_CHUNK = 128
_WPW = _NWORDS // _NW
_CPW = _WPW // _CHUNK
_FIRE = 13


@functools.lru_cache(maxsize=None)
def _make_sc_gather():
    return functools.partial(
        pl.kernel,
        out_type=(
            jax.ShapeDtypeStruct((_NWORDS,), jnp.float32),
            jax.ShapeDtypeStruct((_PAD * _PAD,), jnp.float32),
        ),
        mesh=plsc.VectorSubcoreMesh(core_axis_name="c", subcore_axis_name="s"),
        scratch_types=[
            pltpu.VMEM((_CPW, _CHUNK), jnp.int32),
            pltpu.VMEM((_WPW,), jnp.float32),
            pltpu.VMEM((6, 128), jnp.float32),
            pltpu.VMEM((6, 128), jnp.int32),
            pltpu.SemaphoreType.DMA,
        ],
        compiler_params=pltpu.CompilerParams(use_tc_tiling_on_sc=False),
    )(_sc_gather_body)


def _sc_gather_body(table_hbm, idx_hbm, beta_hbm, bidx_hbm,
                    out_hbm, bmat_hbm, idx_v, rows_v, beta_v, bidx_v, sem):
    wid = lax.axis_index("s") * 2 + lax.axis_index("c")
    pltpu.sync_copy(idx_hbm.at[wid], idx_v)

    @pl.when(wid == 0)
    def _():
        pltpu.sync_copy(beta_hbm, beta_v)
        pltpu.sync_copy(bidx_hbm, bidx_v)
        scps = []
        for j in range(6):
            scps.append(
                pltpu.async_copy(
                    beta_v.at[j], bmat_hbm.at[bidx_v.at[j]], sem,
                )
            )
        for cp in scps:
            cp.wait()

    for g in range(_CPW // _FIRE):
        cps = []
        for j in range(_FIRE):
            jj = g * _FIRE + j
            cps.append(
                pltpu.async_copy(
                    table_hbm.at[idx_v.at[jj]],
                    rows_v.at[pl.ds(jj * _CHUNK, _CHUNK)],
                    sem,
                )
            )
        for cp in cps:
            cp.wait()
    pltpu.sync_copy(rows_v, out_hbm.at[pl.ds(wid * _WPW, _WPW)])


def _dotNT(a, b):
    return lax.dot_general(
        a, b, (((1,), (1,)), ((), ())),
        preferred_element_type=jnp.float32,
        precision=lax.Precision.HIGHEST,
    )


def _tc_body(es_ref, xdt_ref, w_ref, bmat_ref, out_ref):
    dpt = xdt_ref[...] * w_ref[...]
    zpad = jnp.zeros((_PAD - _NF, _B), jnp.float32)
    f = [jnp.concatenate([es_ref[d], dpt, zpad], axis=0) for d in range(_D)]
    s = _dotNT(f[0], f[0])
    for d in range(1, _D):
        s = s + _dotNT(f[d], f[d])
    m2 = s * (1.0 / _B)
    q = None
    for d in range(_D):
        for d2 in range(d, _D):
            g = f[d] * f[d2]
            t = _dotNT(g, g)
            t = t if d == d2 else t * 2.0
            q = t if q is None else q + t
    var = q * (1.0 / _B) - m2 * m2
    ii = lax.broadcasted_iota(jnp.int32, (_PAD, _PAD), 0)
    jj = lax.broadcasted_iota(jnp.int32, (_PAD, _PAD), 1)
    pair_mask = (ii < jj) & (jj < _NF)
    bm = jnp.where(pair_mask, bmat_ref[...], 0.0)
    w = jnp.tanh(bm) * lax.rsqrt(var + _EPS)
    c = jnp.sum(w * m2)
    acc = None
    for d in range(_D):
        a = lax.dot_general(
            w, f[d], (((0,), (0,)), ((), ())),
            preferred_element_type=jnp.float32,
            precision=lax.Precision.HIGHEST,
        ) * f[d]
        acc = a if acc is None else acc + a
    out_ref[...] = jnp.sum(acc, axis=0, keepdims=True) - c


_tc_call = pl.pallas_call(
    _tc_body,
    out_shape=jax.ShapeDtypeStruct((1, _B), jnp.float32),
    compiler_params=pltpu.CompilerParams(vmem_limit_bytes=100 * 1024 * 1024),
)


def kernel(X, emb_tables, weight, beta):
    sparse_idx_t = X[:, :_NS].astype(jnp.int32).T
    table_flat = emb_tables.transpose(0, 2, 1).reshape(_NS * _D * _V)
    widx = (sparse_idx_t[None, :, :]
            + (jnp.arange(_D, dtype=jnp.int32) * _V)[:, None, None]
            + (jnp.arange(_NS, dtype=jnp.int32) * (_D * _V))[None, :, None]
            ).reshape(_NW, _CPW, _CHUNK)
    beta_pad = jnp.pad(beta, (0, _BPAD - _P)).reshape(6, 128)
    es_flat, bmat_flat = _make_sc_gather()(
        table_flat, widx, beta_pad, jnp.asarray(_BIDX))
    es = es_flat.reshape(_D, _NS, _B)
    xdt = X[:, _NS:].T
    out = _tc_call(es, xdt, weight, bmat_flat.reshape(_PAD, _PAD))
    return out.reshape(_B, 1)
